# baseline (speedup 1.0000x reference)
import functools

import jax
import jax.numpy as jnp
from jax.experimental import pallas as pl
from jax.experimental.pallas import tpu as pltpu


def _norms_body(w_ref, h_ref, out_ref, *, rows_per_block, last_step):
    step = pl.program_id(0)

    @pl.when(step == 0)
    def _():
        out_ref[...] = jnp.zeros_like(out_ref)

    s = jnp.abs(w_ref[...]) * h_ref[...]
    sq = s * s

    for k in range(rows_per_block // 8):
        out_ref[...] += sq[8 * k:8 * k + 8, :]

    @pl.when(step == last_step)
    def _():
        acc = out_ref[...]
        a = acc[0:4, :] + acc[4:8, :]
        b = a[0:2, :] + a[2:4, :]
        c = b[0:1, :] + b[1:2, :]
        out_ref[0:1, :] = c


def _column_norm_sums(W, H):
    R, C = W.shape
    br = min(R, max(8, (2 ** 21) // C))
    nb = R // br
    out = pl.pallas_call(
        functools.partial(_norms_body, rows_per_block=br, last_step=nb - 1),
        grid=(nb,),
        in_specs=[
            pl.BlockSpec((br, C), lambda i: (i, 0)),
            pl.BlockSpec((br, C), lambda i: (i, 0)),
        ],
        out_specs=pl.BlockSpec((8, C), lambda i: (0, 0)),
        out_shape=jax.ShapeDtypeStruct((8, C), jnp.float32),
    )(W, H)
    return out[0]


def _rank_body(l2col_ref, l2row_ref, perm_ref, dest_ref, *, bi, C, last_step):
    step = pl.program_id(0)

    @pl.when(step == 0)
    def _():
        perm_ref[...] = jnp.zeros_like(perm_ref)

    mine = l2col_ref[...]
    alln = l2row_ref[...]
    i_idx = step * bi + jax.lax.broadcasted_iota(jnp.int32, (bi, 1), 0)
    j_idx = jax.lax.broadcasted_iota(jnp.int32, (bi, C), 1)
    gt = alln > mine
    tie = (alln == mine) & (j_idx < i_idx)
    rank = jnp.sum((gt | tie).astype(jnp.int32), axis=1, keepdims=True)

    half = C // 2
    u = (C - 1) - rank
    dest = jnp.where(rank < half,
                     2 * rank - (rank & 1),
                     2 * u + 2 - (u & 1))
    dest_ref[...] = dest

    onehot = (dest == j_idx).astype(jnp.int32)
    perm_ref[...] += jnp.sum(onehot * i_idx, axis=0, keepdims=True)


def _rank_and_perm(l2):
    C = l2.shape[0]
    bi = min(C, 512)
    nb = C // bi
    perm, dest = pl.pallas_call(
        functools.partial(_rank_body, bi=bi, C=C, last_step=nb - 1),
        grid=(nb,),
        in_specs=[
            pl.BlockSpec((bi, 1), lambda i: (i, 0)),
            pl.BlockSpec((1, C), lambda i: (0, 0)),
        ],
        out_specs=[
            pl.BlockSpec((1, C), lambda i: (0, 0)),
            pl.BlockSpec((bi, 1), lambda i: (i, 0)),
        ],
        out_shape=[
            jax.ShapeDtypeStruct((1, C), jnp.int32),
            jax.ShapeDtypeStruct((C, 1), jnp.int32),
        ],
    )(l2.reshape(C, 1), l2.reshape(1, C))
    return perm.reshape(C), dest


def _top2of4_mask(vals):
    a = jnp.abs(vals)
    shape = a.shape
    r_vec = jax.lax.broadcasted_iota(jnp.int32, shape, 1) % 4
    rolls = {s: pltpu.roll(a, s % shape[1], 1) for s in range(-3, 4)}

    def window_pos(k):
        p = rolls[3 - k]
        for r in (2, 1, 0):
            p = jnp.where(r_vec == r, rolls[r - k], p)
        return p

    cnt = jnp.zeros(shape, jnp.int32)
    for k in range(4):
        p = window_pos(k)
        beats = (p > a) | ((p == a) & (k < r_vec))
        cnt = cnt + beats.astype(jnp.int32)
    return cnt < 2


def _gather_body(dest_ref, w_ref, out_ref, acc_ref, *, bj, bk, last_k):
    ko = pl.program_id(2)

    @pl.when(ko == 0)
    def _():
        acc_ref[...] = jnp.zeros_like(acc_ref)

    jo = pl.program_id(1)
    dest = dest_ref[...]
    j_glob = jo * bj + jax.lax.broadcasted_iota(jnp.int32, (bk, bj), 1)
    P = (dest == j_glob).astype(jnp.float32)
    acc_ref[...] += jnp.dot(w_ref[...], P, preferred_element_type=jnp.float32,
                            precision=jax.lax.Precision.HIGHEST)

    @pl.when(ko == last_k)
    def _():
        g = acc_ref[...]
        keep = _top2of4_mask(g)
        out_ref[...] = jnp.where(keep, g, 0.0)


def _gather_mask(W, dest):
    R, C = W.shape
    bi = min(R, 512)
    bj = min(C, 512)
    bk = min(C, 512)
    grid = (R // bi, C // bj, C // bk)
    out = pl.pallas_call(
        functools.partial(_gather_body, bj=bj, bk=bk, last_k=grid[2] - 1),
        grid=grid,
        in_specs=[
            pl.BlockSpec((bk, 1), lambda io, jo, ko: (ko, 0)),
            pl.BlockSpec((bi, bk), lambda io, jo, ko: (io, ko)),
        ],
        out_specs=pl.BlockSpec((bi, bj), lambda io, jo, ko: (io, jo)),
        out_shape=jax.ShapeDtypeStruct((R, C), jnp.float32),
        scratch_shapes=[pltpu.VMEM((bi, bj), jnp.float32)],
    )(dest, W)
    return out


def _process_pair(W, H):
    sums = _column_norm_sums(W, H)
    l2 = jnp.sqrt(sums)
    perm, dest = _rank_and_perm(l2)
    w_sparse = _gather_mask(W, dest)
    return w_sparse, perm


def kernel(W0, H0, W2, H2):
    w_sparse0, r1 = _process_pair(W0.astype(jnp.float32), H0.astype(jnp.float32))
    w_sparse2, r2 = _process_pair(W2.astype(jnp.float32), H2.astype(jnp.float32))
    return w_sparse0, w_sparse2, r1, r2
